# Initial kernel scaffold; baseline (speedup 1.0000x reference)
import functools

import jax
import jax.numpy as jnp
import numpy as np
from jax import lax
from jax.experimental import pallas as pl
from jax.experimental.pallas import tpu as pltpu
from jax.experimental.pallas import tpu_sc as plsc

_V_MIN = -10.0
_V_MAX = 10.0
_NUM_ATOMS = 51
_DISCOUNT = 0.99
_ATOM_DELTA = (_V_MAX - _V_MIN) / (_NUM_ATOMS - 1)
_ATOMS_F32 = np.asarray(
    [_V_MIN + _ATOM_DELTA * i for i in range(_NUM_ATOMS)], dtype=np.float32
)

_NC = 2
_NS = 16
_L = 16
_NW = _NC * _NS


@functools.lru_cache(maxsize=None)
def _make_kernel(bs: int, num_atoms: int):
    A = num_atoms
    rows_per_w = bs // _NW
    chunk = min(1024, rows_per_w)
    n_chunks = rows_per_w // chunk
    groups = chunk // _L

    mesh = plsc.VectorSubcoreMesh(
        core_axis_name="c", subcore_axis_name="s",
        num_cores=_NC, num_subcores=_NS,
    )

    @functools.partial(
        pl.kernel,
        out_type=jax.ShapeDtypeStruct((bs * A,), jnp.float32),
        mesh=mesh,
        scratch_types=[
            pltpu.VMEM((rows_per_w,), jnp.float32),
            pltpu.VMEM((rows_per_w,), jnp.float32),
            pltpu.VMEM((chunk * A,), jnp.float32),
            pltpu.VMEM((chunk * A,), jnp.float32),
        ],
    )
    def projection_kernel(rew_hbm, nd_hbm, probs_hbm, out_hbm,
                          rew_v, nd_v, probs_v, acc_v):
        cid = lax.axis_index("c")
        sid = lax.axis_index("s")
        wid = sid * _NC + cid
        row0 = wid * rows_per_w
        pltpu.sync_copy(rew_hbm.at[pl.ds(row0, rows_per_w)], rew_v)
        pltpu.sync_copy(nd_hbm.at[pl.ds(row0, rows_per_w)], nd_v)
        lanes = lax.iota(jnp.int32, _L)
        zeros = jnp.zeros((_L,), jnp.float32)

        def chunk_body(t, carry):
            cbase = t * chunk
            pltpu.sync_copy(
                probs_hbm.at[pl.ds((row0 + cbase) * A, chunk * A)], probs_v)

            def group_body(g, gcarry):
                rbase = g * _L
                r = rew_v[pl.ds(cbase + rbase, _L)]
                nd = nd_v[pl.ds(cbase + rbase, _L)]
                c = _DISCOUNT * nd
                rowb = (rbase + lanes) * A
                rowmax = rowb + (A - 1)
                gb = rbase * A
                for kk in range(A):
                    acc_v[pl.ds(gb + kk * _L, _L)] = zeros
                for j in range(A):
                    pj = plsc.load_gather(probs_v, [rowb + j])
                    z = r + c * float(_ATOMS_F32[j])
                    z = jnp.maximum(z, _V_MIN)
                    z = jnp.minimum(z, _V_MAX)
                    b = (z - _V_MIN) / _ATOM_DELTA
                    li = b.astype(jnp.int32)
                    frac = b - li.astype(jnp.float32)
                    wl = (1.0 - frac) * pj
                    wu = frac * pj
                    idxl = rowb + li
                    idxu = jnp.minimum(idxl + 1, rowmax)
                    plsc.addupdate_scatter(acc_v, [idxl], wl)
                    plsc.addupdate_scatter(acc_v, [idxu], wu)
                return gcarry

            lax.fori_loop(0, groups, group_body, 0)
            pltpu.sync_copy(
                acc_v, out_hbm.at[pl.ds((row0 + cbase) * A, chunk * A)])
            return carry

        lax.fori_loop(0, n_chunks, chunk_body, 0)

    return projection_kernel


def kernel(reward, probs, not_done):
    bs, A = probs.shape
    run = _make_kernel(bs, A)
    out_flat = run(reward.reshape(bs), not_done.reshape(bs),
                   probs.reshape(bs * A))
    return out_flat.reshape(bs, A)


# speedup vs baseline: 62.5767x; 62.5767x over previous
import functools

import jax
import jax.numpy as jnp
import numpy as np
from jax import lax
from jax.experimental import pallas as pl
from jax.experimental.pallas import tpu as pltpu
from jax.experimental.pallas import tpu_sc as plsc

_V_MIN = -10.0
_V_MAX = 10.0
_NUM_ATOMS = 51
_DISCOUNT = 0.99
_ATOM_DELTA = (_V_MAX - _V_MIN) / (_NUM_ATOMS - 1)
_ATOMS_F32 = np.asarray(
    [_V_MIN + _ATOM_DELTA * i for i in range(_NUM_ATOMS)], dtype=np.float32
)

_NC = 2
_NS = 16
_L = 16
_NW = _NC * _NS


@functools.lru_cache(maxsize=None)
def _make_kernel(bs: int, num_atoms: int):
    A = num_atoms
    rows_per_w = bs // _NW
    chunk = min(1024, rows_per_w)
    n_chunks = rows_per_w // chunk
    groups = chunk // _L

    mesh = plsc.VectorSubcoreMesh(
        core_axis_name="c", subcore_axis_name="s",
        num_cores=_NC, num_subcores=_NS,
    )

    @functools.partial(
        pl.kernel,
        out_type=jax.ShapeDtypeStruct((bs * A,), jnp.float32),
        mesh=mesh,
        compiler_params=pltpu.CompilerParams(needs_layout_passes=False),
        scratch_types=[
            pltpu.VMEM((rows_per_w,), jnp.float32),
            pltpu.VMEM((rows_per_w,), jnp.float32),
            pltpu.VMEM((chunk * A,), jnp.float32),
            pltpu.VMEM((chunk * A,), jnp.float32),
        ],
    )
    def projection_kernel(rew_hbm, nd_hbm, probs_hbm, out_hbm,
                          rew_v, nd_v, probs_v, acc_v):
        cid = lax.axis_index("c")
        sid = lax.axis_index("s")
        wid = sid * _NC + cid
        row0 = wid * rows_per_w
        pltpu.sync_copy(rew_hbm.at[pl.ds(row0, rows_per_w)], rew_v)
        pltpu.sync_copy(nd_hbm.at[pl.ds(row0, rows_per_w)], nd_v)
        lanes = lax.iota(jnp.int32, _L)
        zeros = jnp.zeros((_L,), jnp.float32)

        def chunk_body(t, carry):
            cbase = t * chunk
            pltpu.sync_copy(
                probs_hbm.at[pl.ds((row0 + cbase) * A, chunk * A)], probs_v)

            def group_body(g, gcarry):
                rbase = g * _L
                r = rew_v[pl.ds(cbase + rbase, _L)]
                nd = nd_v[pl.ds(cbase + rbase, _L)]
                c = _DISCOUNT * nd
                rowb = (rbase + lanes) * A
                rowmax = rowb + (A - 1)
                gb = rbase * A
                for kk in range(A):
                    acc_v[pl.ds(gb + kk * _L, _L)] = zeros
                for j in range(A):
                    pj = plsc.load_gather(probs_v, [rowb + j])
                    z = r + c * float(_ATOMS_F32[j])
                    z = jnp.maximum(z, _V_MIN)
                    z = jnp.minimum(z, _V_MAX)
                    b = (z - _V_MIN) / _ATOM_DELTA
                    li = b.astype(jnp.int32)
                    frac = b - li.astype(jnp.float32)
                    wl = (1.0 - frac) * pj
                    wu = frac * pj
                    idxl = rowb + li
                    idxu = jnp.minimum(idxl + 1, rowmax)
                    plsc.addupdate_scatter(acc_v, [idxl], wl)
                    plsc.addupdate_scatter(acc_v, [idxu], wu)
                return gcarry

            lax.fori_loop(0, groups, group_body, 0)
            pltpu.sync_copy(
                acc_v, out_hbm.at[pl.ds((row0 + cbase) * A, chunk * A)])
            return carry

        lax.fori_loop(0, n_chunks, chunk_body, 0)

    return projection_kernel


def kernel(reward, probs, not_done):
    bs, A = probs.shape
    run = _make_kernel(bs, A)
    out_flat = run(reward.reshape(bs), not_done.reshape(bs),
                   probs.reshape(bs * A))
    return out_flat.reshape(bs, A)
